# baseline (speedup 1.0000x reference)
import functools

import jax
import jax.numpy as jnp
from jax.experimental import pallas as pl

B, N, K, F, NB = 8, 256, 32, 128, 20
CUTOFF = 5.0
P = 6.0

TN = 64
TNK = TN * K
NT = N // TN


def _poly_cutoff(d):
    x = d * (1.0 / CUTOFF)
    c1 = (P + 1.0) * (P + 2.0) / 2.0
    c2 = P * (P + 2.0)
    c3 = P * (P + 1.0) / 2.0
    x2 = x * x
    x3 = x2 * x
    x6 = x3 * x3
    f = 1.0 - c1 * x6 + c2 * x6 * x - c3 * x6 * x2
    return f * (d < CUTOFF).astype(d.dtype)


def _silu(x):
    return x * jax.nn.sigmoid(x)


def _dot(a, b):
    return jax.lax.dot_general(a, b, (((1,), (0,)), ((), ())),
                               preferred_element_type=jnp.float32)


def _ksum(x2d):
    return jnp.sum(x2d.reshape(TN, K, x2d.shape[-1]), axis=1)


def _mpl_kernel(inv_node_ref, edge_ref, scal_ref,
                eqF_ref, eqf_ref, dr_ref,
                w_ime, b_ime, w_imn1, b_imn1, w_imn2, b_imn2, w_emc,
                w_emf1, b_emf1, w_emf2, b_emf2,
                w_esc1, b_esc1, w_esc2, b_esc2,
                w_eme1, w_eme2,
                w_isc1, b_isc1, w_isc2, b_isc2,
                out_inv_ref, out_eqF_ref, out_eqf_ref, out_dr_ref):
    t = pl.program_id(1)
    row0 = t * TN

    inv_full = inv_node_ref[0]
    dr_full = dr_ref[0]

    imn_full = _dot(_silu(_dot(inv_full, w_imn1[...]) + b_imn1[...]),
                    w_imn2[...]) + b_imn2[...]

    inv_tile = inv_node_ref[0, pl.ds(row0, TN)]
    dr_tile = dr_ref[0, pl.ds(row0, TN)]
    imn_tile = _dot(_silu(_dot(inv_tile, w_imn1[...]) + b_imn1[...]),
                    w_imn2[...]) + b_imn2[...]

    scal = scal_ref[0]
    cut_l = _poly_cutoff(scal[0:1])
    packT = scal[1:6].T
    mask = packT[:, 0:1]
    dvec = packT[:, 1:4]
    nbr = packT[:, 4:5].astype(jnp.int32)
    eqF_tile = eqF_ref[0]
    eqf_tile = eqf_ref[0]

    cols = jax.lax.broadcasted_iota(jnp.int32, (TNK, N), 1)
    onehot = jnp.where(cols == nbr, mask, 0.0).astype(jnp.bfloat16)
    gsrc = jnp.concatenate([imn_full, dr_full], axis=1)
    g_hi = gsrc.astype(jnp.bfloat16)
    g_lo = (gsrc - g_hi.astype(jnp.float32)).astype(jnp.bfloat16)
    gathered = _dot(onehot, g_hi) + _dot(onehot, g_lo)
    imn_g = gathered[:, :F]

    edge_aug = jnp.concatenate([edge_ref[0] * cut_l, cut_l], axis=0)
    w_aug = jnp.concatenate([w_ime[...], b_ime[...]], axis=0)
    ime = jax.lax.dot_general(edge_aug, w_aug,
                              (((0,), (0,)), ((), ())),
                              preferred_element_type=jnp.float32)
    inv_msg3 = (ime.reshape(TN, K, F) * imn_g.reshape(TN, K, F)
                * imn_tile.reshape(TN, 1, F))
    inv_msg = inv_msg3.reshape(TNK, F)

    e_lane = jax.lax.broadcasted_iota(jnp.int32, (TN, TNK), 1)
    n_sub = jax.lax.broadcasted_iota(jnp.int32, (TN, TNK), 0)
    seg = jnp.where((e_lane // K) == n_sub, 1.0, 0.0)

    inv_new = inv_tile + _dot(seg, inv_msg)

    s = _dot(inv_msg, w_emc[...])
    eqF_new = eqF_tile + _ksum(s * dvec)

    bf = jnp.bfloat16
    inv_msg_h = inv_msg.astype(bf)
    emf_h1 = _silu(_dot(inv_msg_h, w_emf1[...].astype(bf)) + b_emf1[...])
    emf = _dot(emf_h1.astype(bf), w_emf2[...].astype(bf)) + b_emf2[...]
    eme_h1 = _silu(_dot(inv_msg_h, w_eme1[...].astype(bf)))
    eme = _dot(eme_h1.astype(bf), w_eme2[...].astype(bf))

    upd_f = []
    upd_dr = []
    for c in range(3):
        wc = s * dvec[:, c:c + 1]
        upd_f.append(_dot(seg, emf * wc))
        dr_g_c = gathered[:, (c + 1) * F:(c + 2) * F]
        upd_dr.append(_dot(seg, eme * dr_g_c))

    eqf_new3 = eqf_tile + jnp.concatenate(
        [u.reshape(TN, 1, F) for u in upd_f], axis=1)

    esc = _dot(_silu(_dot(inv_new, w_esc1[...]) + b_esc1[...]),
               w_esc2[...]) + b_esc2[...]
    isc = _dot(_silu(_dot(inv_new, w_isc1[...]) + b_isc1[...]),
               w_isc2[...]) + b_isc2[...]

    dot_fd = jnp.zeros((TN, F), jnp.float32)
    dr_new = []
    for c in range(3):
        dr_c = dr_tile[:, c * F:(c + 1) * F] + upd_dr[c] + esc * upd_f[c]
        dr_new.append(dr_c.reshape(TN, 1, F))
        dot_fd = dot_fd + eqf_new3[:, c, :] * dr_c

    out_inv_ref[0] = inv_new - isc * dot_fd
    out_eqF_ref[0] = eqF_new
    out_eqf_ref[0] = eqf_new3
    out_dr_ref[0] = jnp.concatenate(dr_new, axis=1)


@jax.jit
def kernel(invariant_node, invariant_edge, distances, distance_vector,
           neighbors, neighbor_mask, equivariant_node_F, equivariant_node_f,
           equivariant_node_dr, params):
    p = params
    NK = N * K
    edgeT = invariant_edge.reshape(B, NK, NB).transpose(0, 2, 1)
    dv_t = distance_vector.reshape(B, NK, 3).transpose(0, 2, 1)
    scal = jnp.concatenate(
        [distances.reshape(B, 1, NK), neighbor_mask.reshape(B, 1, NK),
         dv_t, neighbors.astype(jnp.float32).reshape(B, 1, NK),
         jnp.zeros((B, 2, NK), jnp.float32)], axis=1)
    dr2 = equivariant_node_dr.reshape(B, N, 3 * F)

    def full(a):
        return pl.BlockSpec((1,) + a.shape[1:], lambda b, t: (b, 0, 0))

    def tiled(a, tile):
        return pl.BlockSpec((1, tile, a.shape[2]), lambda b, t: (b, t, 0))

    def wspec(w):
        return pl.BlockSpec(w.shape, lambda b, t: (0,) * w.ndim)

    weights = [p["w_ime"], p["b_ime"].reshape(1, F),
               p["w_imn1"], p["b_imn1"].reshape(1, F),
               p["w_imn2"], p["b_imn2"].reshape(1, F),
               p["w_emc"],
               p["w_emf1"], p["b_emf1"].reshape(1, F),
               p["w_emf2"], p["b_emf2"].reshape(1, F),
               p["w_esc1"], p["b_esc1"].reshape(1, F),
               p["w_esc2"], p["b_esc2"].reshape(1, F),
               p["w_eme1"], p["w_eme2"],
               p["w_isc1"], p["b_isc1"].reshape(1, F),
               p["w_isc2"], p["b_isc2"].reshape(1, F)]

    edge_spec = pl.BlockSpec((1, NB, TNK), lambda b, t: (b, 0, t))
    scal_spec = pl.BlockSpec((1, 8, TNK), lambda b, t: (b, 0, t))
    spec4 = pl.BlockSpec((1, TN, 3, F), lambda b, t: (b, t, 0, 0))
    in_specs = ([full(invariant_node), edge_spec, scal_spec,
                 tiled(equivariant_node_F, TN), spec4, full(dr2)]
                + [wspec(w) for w in weights])

    out_shapes = [jax.ShapeDtypeStruct((B, N, F), jnp.float32),
                  jax.ShapeDtypeStruct((B, N, 3), jnp.float32),
                  jax.ShapeDtypeStruct((B, N, 3, F), jnp.float32),
                  jax.ShapeDtypeStruct((B, N, 3, F), jnp.float32)]
    out_specs = [tiled(out_shapes[0], TN), tiled(out_shapes[1], TN),
                 spec4, spec4]

    outs = pl.pallas_call(
        _mpl_kernel,
        grid=(B, NT),
        in_specs=in_specs,
        out_specs=out_specs,
        out_shape=out_shapes,
    )(invariant_node, edgeT, scal,
      equivariant_node_F, equivariant_node_f, dr2, *weights)

    inv_new, eqF_new, eqf_new, dr_new = outs
    return (inv_new, eqF_new, eqf_new, dr_new)


# speedup vs baseline: 1.0515x; 1.0515x over previous
import functools

import jax
import jax.numpy as jnp
from jax.experimental import pallas as pl

B, N, K, F, NB = 8, 256, 32, 128, 20
CUTOFF = 5.0
P = 6.0

TN = 128
TNK = TN * K
NT = N // TN


def _poly_cutoff(d):
    x = d * (1.0 / CUTOFF)
    c1 = (P + 1.0) * (P + 2.0) / 2.0
    c2 = P * (P + 2.0)
    c3 = P * (P + 1.0) / 2.0
    x2 = x * x
    x3 = x2 * x
    x6 = x3 * x3
    f = 1.0 - c1 * x6 + c2 * x6 * x - c3 * x6 * x2
    return f * (d < CUTOFF).astype(d.dtype)


def _silu(x):
    return x * jax.nn.sigmoid(x)


def _dot(a, b):
    return jax.lax.dot_general(a, b, (((1,), (0,)), ((), ())),
                               preferred_element_type=jnp.float32)


def _ksum(x2d):
    return jnp.sum(x2d.reshape(TN, K, x2d.shape[-1]), axis=1)


def _mpl_kernel(inv_node_ref, edge_ref, scal_ref,
                eqF_ref, eqf_ref, dr_ref,
                w_ime, b_ime, w_imn1, b_imn1, w_imn2, b_imn2, w_emc,
                w_emf1, b_emf1, w_emf2, b_emf2,
                w_esc1, b_esc1, w_esc2, b_esc2,
                w_eme1, w_eme2,
                w_isc1, b_isc1, w_isc2, b_isc2,
                out_inv_ref, out_eqF_ref, out_eqf_ref, out_dr_ref):
    t = pl.program_id(1)
    row0 = t * TN

    inv_full = inv_node_ref[0]
    dr_full = dr_ref[0]

    imn_full = _dot(_silu(_dot(inv_full, w_imn1[...]) + b_imn1[...]),
                    w_imn2[...]) + b_imn2[...]

    inv_tile = inv_node_ref[0, pl.ds(row0, TN)]
    dr_tile = dr_ref[0, pl.ds(row0, TN)]
    imn_tile = _dot(_silu(_dot(inv_tile, w_imn1[...]) + b_imn1[...]),
                    w_imn2[...]) + b_imn2[...]

    scal = scal_ref[0]
    cut_l = _poly_cutoff(scal[0:1])
    packT = scal[1:6].T
    mask = packT[:, 0:1]
    dvec = packT[:, 1:4]
    nbr = packT[:, 4:5].astype(jnp.int32)
    eqF_tile = eqF_ref[0]
    eqf_tile = eqf_ref[0]

    cols = jax.lax.broadcasted_iota(jnp.int32, (TNK, N), 1)
    onehot = jnp.where(cols == nbr, mask, 0.0).astype(jnp.bfloat16)
    gsrc = jnp.concatenate([imn_full, dr_full], axis=1)
    g_hi = gsrc.astype(jnp.bfloat16)
    g_lo = (gsrc - g_hi.astype(jnp.float32)).astype(jnp.bfloat16)
    gathered = _dot(onehot, g_hi) + _dot(onehot, g_lo)
    imn_g = gathered[:, :F]

    edge_aug = jnp.concatenate([edge_ref[0] * cut_l, cut_l], axis=0)
    w_aug = jnp.concatenate([w_ime[...], b_ime[...]], axis=0)
    ime = jax.lax.dot_general(edge_aug, w_aug,
                              (((0,), (0,)), ((), ())),
                              preferred_element_type=jnp.float32)
    inv_msg3 = (ime.reshape(TN, K, F) * imn_g.reshape(TN, K, F)
                * imn_tile.reshape(TN, 1, F))
    inv_msg = inv_msg3.reshape(TNK, F)

    inv_new = inv_tile + _ksum(inv_msg)

    s = _dot(inv_msg, w_emc[...])
    eqF_new = eqF_tile + _ksum(s * dvec)

    bf = jnp.bfloat16
    inv_msg_h = inv_msg.astype(bf)
    emf_h1 = _silu(_dot(inv_msg_h, w_emf1[...].astype(bf)) + b_emf1[...])
    emf = _dot(emf_h1.astype(bf), w_emf2[...].astype(bf)) + b_emf2[...]
    eme_h1 = _silu(_dot(inv_msg_h, w_eme1[...].astype(bf)))
    eme = _dot(eme_h1.astype(bf), w_eme2[...].astype(bf))

    upd_f = []
    upd_dr = []
    for c in range(3):
        wc = s * dvec[:, c:c + 1]
        upd_f.append(_ksum(emf * wc))
        dr_g_c = gathered[:, (c + 1) * F:(c + 2) * F]
        upd_dr.append(_ksum(eme * dr_g_c))

    eqf_new3 = eqf_tile + jnp.concatenate(
        [u.reshape(TN, 1, F) for u in upd_f], axis=1)

    esc = _dot(_silu(_dot(inv_new, w_esc1[...]) + b_esc1[...]),
               w_esc2[...]) + b_esc2[...]
    isc = _dot(_silu(_dot(inv_new, w_isc1[...]) + b_isc1[...]),
               w_isc2[...]) + b_isc2[...]

    dot_fd = jnp.zeros((TN, F), jnp.float32)
    dr_new = []
    for c in range(3):
        dr_c = dr_tile[:, c * F:(c + 1) * F] + upd_dr[c] + esc * upd_f[c]
        dr_new.append(dr_c.reshape(TN, 1, F))
        dot_fd = dot_fd + eqf_new3[:, c, :] * dr_c

    out_inv_ref[0] = inv_new - isc * dot_fd
    out_eqF_ref[0] = eqF_new
    out_eqf_ref[0] = eqf_new3
    out_dr_ref[0] = jnp.concatenate(dr_new, axis=1)


@jax.jit
def kernel(invariant_node, invariant_edge, distances, distance_vector,
           neighbors, neighbor_mask, equivariant_node_F, equivariant_node_f,
           equivariant_node_dr, params):
    p = params
    NK = N * K
    edgeT = invariant_edge.reshape(B, NK, NB).transpose(0, 2, 1)
    dv_t = distance_vector.reshape(B, NK, 3).transpose(0, 2, 1)
    scal = jnp.concatenate(
        [distances.reshape(B, 1, NK), neighbor_mask.reshape(B, 1, NK),
         dv_t, neighbors.astype(jnp.float32).reshape(B, 1, NK),
         jnp.zeros((B, 2, NK), jnp.float32)], axis=1)
    dr2 = equivariant_node_dr.reshape(B, N, 3 * F)

    def full(a):
        return pl.BlockSpec((1,) + a.shape[1:], lambda b, t: (b, 0, 0))

    def tiled(a, tile):
        return pl.BlockSpec((1, tile, a.shape[2]), lambda b, t: (b, t, 0))

    def wspec(w):
        return pl.BlockSpec(w.shape, lambda b, t: (0,) * w.ndim)

    weights = [p["w_ime"], p["b_ime"].reshape(1, F),
               p["w_imn1"], p["b_imn1"].reshape(1, F),
               p["w_imn2"], p["b_imn2"].reshape(1, F),
               p["w_emc"],
               p["w_emf1"], p["b_emf1"].reshape(1, F),
               p["w_emf2"], p["b_emf2"].reshape(1, F),
               p["w_esc1"], p["b_esc1"].reshape(1, F),
               p["w_esc2"], p["b_esc2"].reshape(1, F),
               p["w_eme1"], p["w_eme2"],
               p["w_isc1"], p["b_isc1"].reshape(1, F),
               p["w_isc2"], p["b_isc2"].reshape(1, F)]

    edge_spec = pl.BlockSpec((1, NB, TNK), lambda b, t: (b, 0, t))
    scal_spec = pl.BlockSpec((1, 8, TNK), lambda b, t: (b, 0, t))
    spec4 = pl.BlockSpec((1, TN, 3, F), lambda b, t: (b, t, 0, 0))
    in_specs = ([full(invariant_node), edge_spec, scal_spec,
                 tiled(equivariant_node_F, TN), spec4, full(dr2)]
                + [wspec(w) for w in weights])

    out_shapes = [jax.ShapeDtypeStruct((B, N, F), jnp.float32),
                  jax.ShapeDtypeStruct((B, N, 3), jnp.float32),
                  jax.ShapeDtypeStruct((B, N, 3, F), jnp.float32),
                  jax.ShapeDtypeStruct((B, N, 3, F), jnp.float32)]
    out_specs = [tiled(out_shapes[0], TN), tiled(out_shapes[1], TN),
                 spec4, spec4]

    outs = pl.pallas_call(
        _mpl_kernel,
        grid=(B, NT),
        in_specs=in_specs,
        out_specs=out_specs,
        out_shape=out_shapes,
    )(invariant_node, edgeT, scal,
      equivariant_node_F, equivariant_node_f, dr2, *weights)

    inv_new, eqF_new, eqf_new, dr_new = outs
    return (inv_new, eqF_new, eqf_new, dr_new)


# speedup vs baseline: 1.0524x; 1.0009x over previous
import jax
import jax.numpy as jnp
from jax.experimental import pallas as pl

B, N, K, F, NB = 8, 256, 32, 128, 20
CUTOFF = 5.0
P = 6.0

TN = 128
TNK = TN * K
NT = N // TN


def _poly_cutoff(d):
    x = d * (1.0 / CUTOFF)
    c1 = (P + 1.0) * (P + 2.0) / 2.0
    c2 = P * (P + 2.0)
    c3 = P * (P + 1.0) / 2.0
    x2 = x * x
    x3 = x2 * x
    x6 = x3 * x3
    f = 1.0 - c1 * x6 + c2 * x6 * x - c3 * x6 * x2
    return f * (d < CUTOFF).astype(d.dtype)


def _silu(x):
    return x * jax.nn.sigmoid(x)


def _dot(a, b):
    return jax.lax.dot_general(a, b, (((1,), (0,)), ((), ())),
                               preferred_element_type=jnp.float32)


def _ksum(x2d):
    return jnp.sum(x2d.reshape(TN, K, x2d.shape[-1]), axis=1)


def _mpl_kernel(inv_node_ref, edge_ref, scal_ref,
                eqF_ref, eqf_ref, dr_ref,
                w_ime, b_ime, w_imn1, b_imn1, w_imn2, b_imn2, w_emc,
                w_emf1, b_emf1, w_emf2, b_emf2,
                w_esc1, b_esc1, w_esc2, b_esc2,
                w_eme1, w_eme2,
                w_isc1, b_isc1, w_isc2, b_isc2,
                out_inv_ref, out_eqF_ref, out_eqf_ref, out_dr_ref):
    t = pl.program_id(1)
    row0 = t * TN

    inv_full = inv_node_ref[0]
    dr_full = dr_ref[0]

    imn_full = _dot(_silu(_dot(inv_full, w_imn1[...]) + b_imn1[...]),
                    w_imn2[...]) + b_imn2[...]

    inv_tile = inv_node_ref[0, pl.ds(row0, TN)]
    dr_tile = dr_ref[0, pl.ds(row0, TN)]
    imn_tile = _dot(_silu(_dot(inv_tile, w_imn1[...]) + b_imn1[...]),
                    w_imn2[...]) + b_imn2[...]

    scal = scal_ref[0]
    cut_l = _poly_cutoff(scal[0:1])
    packT = scal[1:6].T
    mask = packT[:, 0:1]
    dvec = packT[:, 1:4]
    nbr = packT[:, 4:5].astype(jnp.int32)
    eqF_tile = eqF_ref[0]
    eqf_tile = eqf_ref[0]

    cols = jax.lax.broadcasted_iota(jnp.int32, (TNK, N), 1)
    onehot = jnp.where(cols == nbr, mask, 0.0).astype(jnp.bfloat16)
    gsrc = jnp.concatenate([imn_full, dr_full], axis=1)
    g_hi = gsrc.astype(jnp.bfloat16)
    g_lo = (gsrc - g_hi.astype(jnp.float32)).astype(jnp.bfloat16)
    gathered = _dot(onehot, g_hi) + _dot(onehot, g_lo)
    imn_g = gathered[:, :F]

    edge_aug = jnp.concatenate([edge_ref[0] * cut_l, cut_l], axis=0)
    w_aug = jnp.concatenate([w_ime[...], b_ime[...]], axis=0)
    ime = jax.lax.dot_general(edge_aug, w_aug,
                              (((0,), (0,)), ((), ())),
                              preferred_element_type=jnp.float32)
    inv_msg3 = (ime.reshape(TN, K, F) * imn_g.reshape(TN, K, F)
                * imn_tile.reshape(TN, 1, F))
    inv_msg = inv_msg3.reshape(TNK, F)

    inv_new = inv_tile + _ksum(inv_msg)

    s = _dot(inv_msg, w_emc[...])
    eqF_new = eqF_tile + _ksum(s * dvec)

    bf = jnp.bfloat16
    inv_msg_h = inv_msg.astype(bf)
    emf_h1 = _silu(_dot(inv_msg_h, w_emf1[...].astype(bf)) + b_emf1[...])
    emf = _dot(emf_h1.astype(bf), w_emf2[...].astype(bf)) + b_emf2[...]
    eme_h1 = _silu(_dot(inv_msg_h, w_eme1[...].astype(bf)))
    eme = _dot(eme_h1.astype(bf), w_eme2[...].astype(bf))

    upd_f = []
    upd_dr = []
    for c in range(3):
        wc = s * dvec[:, c:c + 1]
        upd_f.append(_ksum(emf * wc))
        dr_g_c = gathered[:, (c + 1) * F:(c + 2) * F]
        upd_dr.append(_ksum(eme * dr_g_c))

    eqf_new3 = eqf_tile + jnp.concatenate(
        [u.reshape(TN, 1, F) for u in upd_f], axis=1)

    esc = _dot(_silu(_dot(inv_new, w_esc1[...]) + b_esc1[...]),
               w_esc2[...]) + b_esc2[...]
    isc = _dot(_silu(_dot(inv_new, w_isc1[...]) + b_isc1[...]),
               w_isc2[...]) + b_isc2[...]

    dot_fd = jnp.zeros((TN, F), jnp.float32)
    dr_new = []
    for c in range(3):
        dr_c = dr_tile[:, c * F:(c + 1) * F] + upd_dr[c] + esc * upd_f[c]
        dr_new.append(dr_c.reshape(TN, 1, F))
        dot_fd = dot_fd + eqf_new3[:, c, :] * dr_c

    out_inv_ref[0] = inv_new - isc * dot_fd
    out_eqF_ref[0] = eqF_new
    out_eqf_ref[0] = eqf_new3
    out_dr_ref[0] = jnp.concatenate(dr_new, axis=1)


@jax.jit
def kernel(invariant_node, invariant_edge, distances, distance_vector,
           neighbors, neighbor_mask, equivariant_node_F, equivariant_node_f,
           equivariant_node_dr, params):
    p = params
    NK = N * K
    edgeT = invariant_edge.reshape(B, NK, NB).transpose(0, 2, 1)
    dv_t = distance_vector.reshape(B, NK, 3).transpose(0, 2, 1)
    scal = jnp.concatenate(
        [distances.reshape(B, 1, NK), neighbor_mask.reshape(B, 1, NK),
         dv_t, neighbors.astype(jnp.float32).reshape(B, 1, NK),
         jnp.zeros((B, 2, NK), jnp.float32)], axis=1)
    dr2 = equivariant_node_dr.reshape(B, N, 3 * F)

    def full(a):
        return pl.BlockSpec((1,) + a.shape[1:], lambda b, t: (b, 0, 0))

    def tiled(a, tile):
        return pl.BlockSpec((1, tile, a.shape[2]), lambda b, t: (b, t, 0))

    def wspec(w):
        return pl.BlockSpec(w.shape, lambda b, t: (0,) * w.ndim)

    weights = [p["w_ime"], p["b_ime"].reshape(1, F),
               p["w_imn1"], p["b_imn1"].reshape(1, F),
               p["w_imn2"], p["b_imn2"].reshape(1, F),
               p["w_emc"],
               p["w_emf1"], p["b_emf1"].reshape(1, F),
               p["w_emf2"], p["b_emf2"].reshape(1, F),
               p["w_esc1"], p["b_esc1"].reshape(1, F),
               p["w_esc2"], p["b_esc2"].reshape(1, F),
               p["w_eme1"], p["w_eme2"],
               p["w_isc1"], p["b_isc1"].reshape(1, F),
               p["w_isc2"], p["b_isc2"].reshape(1, F)]

    edge_spec = pl.BlockSpec((1, NB, TNK), lambda b, t: (b, 0, t))
    scal_spec = pl.BlockSpec((1, 8, TNK), lambda b, t: (b, 0, t))
    spec4 = pl.BlockSpec((1, TN, 3, F), lambda b, t: (b, t, 0, 0))
    in_specs = ([full(invariant_node), edge_spec, scal_spec,
                 tiled(equivariant_node_F, TN), spec4, full(dr2)]
                + [wspec(w) for w in weights])

    out_shapes = [jax.ShapeDtypeStruct((B, N, F), jnp.float32),
                  jax.ShapeDtypeStruct((B, N, 3), jnp.float32),
                  jax.ShapeDtypeStruct((B, N, 3, F), jnp.float32),
                  jax.ShapeDtypeStruct((B, N, 3, F), jnp.float32)]
    out_specs = [tiled(out_shapes[0], TN), tiled(out_shapes[1], TN),
                 spec4, spec4]

    outs = pl.pallas_call(
        _mpl_kernel,
        grid=(B, NT),
        in_specs=in_specs,
        out_specs=out_specs,
        out_shape=out_shapes,
    )(invariant_node, edgeT, scal,
      equivariant_node_F, equivariant_node_f, dr2, *weights)

    inv_new, eqF_new, eqf_new, dr_new = outs
    return (inv_new, eqF_new, eqf_new, dr_new)
